# Initial kernel scaffold; baseline (speedup 1.0000x reference)
import jax
import jax.numpy as jnp
from jax.experimental import pallas as pl


def kernel(C_masked, W_proj, b_proj, W_ih_f, W_hh_f, b_ih_f, b_hh_f, W_ih_r, W_hh_r, b_ih_r, b_hh_r, W_gcn1, W_gcn2):
    raise NotImplementedError("write your pallas kernel here")


# speedup vs baseline: 4.0093x; 4.0093x over previous
import jax
import jax.numpy as jnp
import numpy as np
from jax.experimental import pallas as pl
from jax.experimental.pallas import tpu as pltpu

S = 64
D = 64
HID = 128
G3 = 192
KNN = 8


def _gru_kernel(c_ref, wp_ref, bp_ref,
                wif_ref, bif_ref, whf_ref, bhf_ref,
                wir_ref, bir_ref, whr_ref, bhr_ref,
                out_ref, cp_ref):
    f32 = jnp.float32
    bb = c_ref.shape[1]
    cp = jnp.dot(c_ref[:].reshape(S * bb, D), wp_ref[:],
                 preferred_element_type=f32) + bp_ref[:]
    cp_ref[:] = cp.reshape(S, bb, HID)
    wif = wif_ref[:]
    bif = bif_ref[:]
    whf = whf_ref[:]
    bhf = bhf_ref[:]
    wir = wir_ref[:]
    bir = bir_ref[:]
    whr = whr_ref[:]
    bhr = bhr_ref[:]

    def step(i, carry):
        hf, hb = carry
        j = S - 1 - i
        xf = cp_ref[pl.ds(i, 1)].reshape(bb, HID)
        gx = jnp.dot(xf, wif, preferred_element_type=f32) + bif
        gh = jnp.dot(hf, whf, preferred_element_type=f32) + bhf
        r = jax.nn.sigmoid(gx[:, 0:64] + gh[:, 0:64])
        z = jax.nn.sigmoid(gx[:, 64:128] + gh[:, 64:128])
        n = jnp.tanh(gx[:, 128:192] + r * gh[:, 128:192])
        hf = (1.0 - z) * n + z * hf
        out_ref[pl.ds(i, 1), :, 0:64] = hf[None]
        xb = cp_ref[pl.ds(j, 1)].reshape(bb, HID)
        gxb = jnp.dot(xb, wir, preferred_element_type=f32) + bir
        ghb = jnp.dot(hb, whr, preferred_element_type=f32) + bhr
        rb = jax.nn.sigmoid(gxb[:, 0:64] + ghb[:, 0:64])
        zb = jax.nn.sigmoid(gxb[:, 64:128] + ghb[:, 64:128])
        nb = jnp.tanh(gxb[:, 128:192] + rb * ghb[:, 128:192])
        hb = (1.0 - zb) * nb + zb * hb
        out_ref[pl.ds(j, 1), :, 64:128] = hb[None]
        return hf, hb

    h0 = jnp.zeros((bb, 64), f32)
    jax.lax.fori_loop(0, S, step, (h0, h0))


def _graph_kernel(h_ref, w1_ref, w2_ref, out_ref):
    f32 = jnp.float32
    h = h_ref[:]
    bb = h.shape[0]
    nrm = jnp.sqrt(jnp.sum(h * h, axis=-1, keepdims=True))
    hn = h / jnp.maximum(nrm, 1e-12)
    dn_sim = (((2,), (2,)), ((0,), (0,)))
    sim = jax.lax.dot_general(hn, hn, dn_sim, preferred_element_type=f32)
    iota_m = jax.lax.broadcasted_iota(jnp.int32, (bb, S, S), 2)
    iota_n = jax.lax.broadcasted_iota(jnp.int32, (bb, S, S), 1)
    a = (iota_m == iota_n).astype(f32)
    simc = sim
    for _ in range(KNN):
        m = jnp.max(simc, axis=-1, keepdims=True)
        oh = simc == m
        a = a + oh.astype(f32)
        simc = jnp.where(oh, -3.0, simc)
    deg = jnp.sum(a, axis=-1, keepdims=True) + 1e-8
    dinv = jax.lax.rsqrt(deg)
    eye = (jax.lax.broadcasted_iota(jnp.int32, (S, S), 0) ==
           jax.lax.broadcasted_iota(jnp.int32, (S, S), 1)).astype(f32)
    dinv_m = jax.lax.dot_general(
        dinv, eye, (((1,), (0,)), ((), ())),
        preferred_element_type=f32,
        precision=jax.lax.Precision.HIGHEST)
    a_norm = (dinv * a) * dinv_m
    dn_a = (((2,), (1,)), ((0,), (0,)))
    hw1 = jnp.dot(h.reshape(bb * S, HID), w1_ref[:],
                  preferred_element_type=f32).reshape(bb, S, HID)
    y1 = jax.lax.dot_general(a_norm, hw1, dn_a, preferred_element_type=f32)
    z1 = jnp.maximum(y1, 0.0)
    zw2 = jnp.dot(z1.reshape(bb * S, HID), w2_ref[:],
                  preferred_element_type=f32).reshape(bb, S, HID)
    out_ref[:] = jax.lax.dot_general(a_norm, zw2, dn_a,
                                     preferred_element_type=f32)


def kernel(C_masked, W_proj, b_proj, W_ih_f, W_hh_f, b_ih_f, b_hh_f,
           W_ih_r, W_hh_r, b_ih_r, b_hh_r, W_gcn1, W_gcn2):
    f32 = jnp.float32
    B, E = C_masked.shape
    bb1 = min(256, B)
    bb2 = min(64, B)

    C_sm = jnp.transpose(C_masked.reshape(B, S, D), (1, 0, 2))
    wifT = W_ih_f.T
    wirT = W_ih_r.T
    whfT = W_hh_f.T
    whrT = W_hh_r.T

    wspec = lambda shape: pl.BlockSpec(shape, lambda i: (0, 0))
    h_sm = pl.pallas_call(
        _gru_kernel,
        grid=(B // bb1,),
        in_specs=[
            pl.BlockSpec((S, bb1, D), lambda i: (0, i, 0)),
            wspec((D, HID)), wspec((1, HID)),
            wspec((HID, G3)), wspec((1, G3)), wspec((64, G3)), wspec((1, G3)),
            wspec((HID, G3)), wspec((1, G3)), wspec((64, G3)), wspec((1, G3)),
        ],
        out_specs=pl.BlockSpec((S, bb1, HID), lambda i: (0, i, 0)),
        out_shape=jax.ShapeDtypeStruct((S, B, HID), f32),
        scratch_shapes=[pltpu.VMEM((S, bb1, HID), f32)],
        compiler_params=pltpu.CompilerParams(
            dimension_semantics=("arbitrary",),
            vmem_limit_bytes=56 * 1024 * 1024,
        ),
    )(C_sm, W_proj, b_proj[None], wifT, b_ih_f[None], whfT, b_hh_f[None],
      wirT, b_ih_r[None], whrT, b_hh_r[None])

    h_bm = jnp.transpose(h_sm, (1, 0, 2))

    z = pl.pallas_call(
        _graph_kernel,
        grid=(B // bb2,),
        in_specs=[
            pl.BlockSpec((bb2, S, HID), lambda i: (i, 0, 0)),
            wspec((HID, HID)), wspec((HID, HID)),
        ],
        out_specs=pl.BlockSpec((bb2, S, HID), lambda i: (i, 0, 0)),
        out_shape=jax.ShapeDtypeStruct((B, S, HID), f32),
        compiler_params=pltpu.CompilerParams(
            dimension_semantics=("arbitrary",),
            vmem_limit_bytes=56 * 1024 * 1024,
        ),
    )(h_bm, W_gcn1, W_gcn2)
    return z


# speedup vs baseline: 4.7694x; 1.1896x over previous
import jax
import jax.numpy as jnp
import numpy as np
from jax.experimental import pallas as pl
from jax.experimental.pallas import tpu as pltpu

S = 64
D = 64
HID = 128
G3 = 192
KNN = 8


def _gru_kernel(c_ref, wp_ref, bp_ref,
                wif_ref, bif_ref, whf_ref, bhf_ref,
                wir_ref, bir_ref, whr_ref, bhr_ref,
                out_ref, cp_ref):
    f32 = jnp.float32
    bb = c_ref.shape[0]
    c_sm = jnp.swapaxes(c_ref[:], 0, 1)
    cp = jnp.dot(c_sm.reshape(S * bb, D), wp_ref[:],
                 preferred_element_type=f32) + bp_ref[:]
    cp_ref[:] = cp.reshape(S, bb, HID)
    wif = wif_ref[:]
    wir = wir_ref[:]
    whf = whf_ref[:]
    whr = whr_ref[:]
    zx = jnp.zeros((HID, 64), f32)
    zh = jnp.zeros((64, 64), f32)
    wx = jnp.concatenate([
        jnp.concatenate([wif[:, 0:64], zx, wif[:, 64:128], zx,
                         wif[:, 128:192], zx], axis=1),
        jnp.concatenate([zx, wir[:, 0:64], zx, wir[:, 64:128],
                         zx, wir[:, 128:192]], axis=1)], axis=0)
    wh = jnp.concatenate([
        jnp.concatenate([whf[:, 0:64], zh, whf[:, 64:128], zh,
                         whf[:, 128:192], zh], axis=1),
        jnp.concatenate([zh, whr[:, 0:64], zh, whr[:, 64:128],
                         zh, whr[:, 128:192]], axis=1)], axis=0)
    bif = bif_ref[:]
    bir = bir_ref[:]
    bhf = bhf_ref[:]
    bhr = bhr_ref[:]
    bx = jnp.concatenate([bif[:, 0:64], bir[:, 0:64], bif[:, 64:128],
                          bir[:, 64:128], bif[:, 128:192],
                          bir[:, 128:192]], axis=1)
    bh = jnp.concatenate([bhf[:, 0:64], bhr[:, 0:64], bhf[:, 64:128],
                          bhr[:, 64:128], bhf[:, 128:192],
                          bhr[:, 128:192]], axis=1)

    def step(i, h):
        j = S - 1 - i
        xf = cp_ref[pl.ds(i, 1)].reshape(bb, HID)
        xb = cp_ref[pl.ds(j, 1)].reshape(bb, HID)
        xc = jnp.concatenate([xf, xb], axis=1)
        gx = jnp.dot(xc, wx, preferred_element_type=f32) + bx
        gh = jnp.dot(h, wh, preferred_element_type=f32) + bh
        r = jax.nn.sigmoid(gx[:, 0:128] + gh[:, 0:128])
        zg = jax.nn.sigmoid(gx[:, 128:256] + gh[:, 128:256])
        n = jnp.tanh(gx[:, 256:384] + r * gh[:, 256:384])
        h = (1.0 - zg) * n + zg * h
        out_ref[pl.ds(i, 1), :, 0:64] = h[None, :, 0:64]
        out_ref[pl.ds(j, 1), :, 64:128] = h[None, :, 64:128]
        return h

    h0 = jnp.zeros((bb, HID), f32)
    jax.lax.fori_loop(0, S, step, h0, unroll=2)


def _graph_kernel(h_ref, w1_ref, w2_ref, out_ref):
    f32 = jnp.float32
    h = jnp.swapaxes(h_ref[:], 0, 1)
    bb = h.shape[0]
    nrm = jnp.sqrt(jnp.sum(h * h, axis=-1, keepdims=True))
    hn = h / jnp.maximum(nrm, 1e-12)
    dn_sim = (((2,), (2,)), ((0,), (0,)))
    sim = jax.lax.dot_general(hn, hn, dn_sim, preferred_element_type=f32)
    simc = sim
    for _ in range(KNN):
        m = jnp.max(simc, axis=-1, keepdims=True)
        simc = jnp.where(simc == m, -3.0, simc)
    eye = (jax.lax.broadcasted_iota(jnp.int32, (bb, S, S), 2) ==
           jax.lax.broadcasted_iota(jnp.int32, (bb, S, S), 1))
    a = (simc != sim).astype(f32) + eye.astype(f32)
    deg = jnp.sum(a, axis=-1, keepdims=True) + 1e-8
    dinv = jax.lax.rsqrt(deg)
    eye = (jax.lax.broadcasted_iota(jnp.int32, (S, S), 0) ==
           jax.lax.broadcasted_iota(jnp.int32, (S, S), 1)).astype(f32)
    dinv_m = jax.lax.dot_general(
        dinv, eye, (((1,), (0,)), ((), ())),
        preferred_element_type=f32,
        precision=jax.lax.Precision.HIGHEST)
    a_norm = (dinv * a) * dinv_m
    dn_a = (((2,), (1,)), ((0,), (0,)))
    hw1 = jnp.dot(h.reshape(bb * S, HID), w1_ref[:],
                  preferred_element_type=f32).reshape(bb, S, HID)
    y1 = jax.lax.dot_general(a_norm, hw1, dn_a, preferred_element_type=f32)
    z1 = jnp.maximum(y1, 0.0)
    zw2 = jnp.dot(z1.reshape(bb * S, HID), w2_ref[:],
                  preferred_element_type=f32).reshape(bb, S, HID)
    out_ref[:] = jax.lax.dot_general(a_norm, zw2, dn_a,
                                     preferred_element_type=f32)


def kernel(C_masked, W_proj, b_proj, W_ih_f, W_hh_f, b_ih_f, b_hh_f,
           W_ih_r, W_hh_r, b_ih_r, b_hh_r, W_gcn1, W_gcn2):
    f32 = jnp.float32
    B, E = C_masked.shape
    bb1 = min(256, B)
    bb2 = min(64, B)

    C_bm = C_masked.reshape(B, S, D)
    wifT = W_ih_f.T
    wirT = W_ih_r.T
    whfT = W_hh_f.T
    whrT = W_hh_r.T

    wspec = lambda shape: pl.BlockSpec(shape, lambda i: (0, 0))
    h_sm = pl.pallas_call(
        _gru_kernel,
        grid=(B // bb1,),
        in_specs=[
            pl.BlockSpec((bb1, S, D), lambda i: (i, 0, 0)),
            wspec((D, HID)), wspec((1, HID)),
            wspec((HID, G3)), wspec((1, G3)), wspec((64, G3)), wspec((1, G3)),
            wspec((HID, G3)), wspec((1, G3)), wspec((64, G3)), wspec((1, G3)),
        ],
        out_specs=pl.BlockSpec((S, bb1, HID), lambda i: (0, i, 0)),
        out_shape=jax.ShapeDtypeStruct((S, B, HID), f32),
        scratch_shapes=[pltpu.VMEM((S, bb1, HID), f32)],
        compiler_params=pltpu.CompilerParams(
            dimension_semantics=("arbitrary",),
            vmem_limit_bytes=56 * 1024 * 1024,
        ),
    )(C_bm, W_proj, b_proj[None], wifT, b_ih_f[None], whfT, b_hh_f[None],
      wirT, b_ih_r[None], whrT, b_hh_r[None])

    z = pl.pallas_call(
        _graph_kernel,
        grid=(B // bb2,),
        in_specs=[
            pl.BlockSpec((S, bb2, HID), lambda i: (0, i, 0)),
            wspec((HID, HID)), wspec((HID, HID)),
        ],
        out_specs=pl.BlockSpec((bb2, S, HID), lambda i: (i, 0, 0)),
        out_shape=jax.ShapeDtypeStruct((B, S, HID), f32),
        compiler_params=pltpu.CompilerParams(
            dimension_semantics=("arbitrary",),
            vmem_limit_bytes=56 * 1024 * 1024,
        ),
    )(h_sm, W_gcn1, W_gcn2)
    return z


# speedup vs baseline: 5.1287x; 1.0753x over previous
import jax
import jax.numpy as jnp
import numpy as np
from jax.experimental import pallas as pl
from jax.experimental.pallas import tpu as pltpu

S = 64
D = 64
HID = 128
G3 = 192
KNN = 8


def _gru_kernel(c_ref, wp_ref, bp_ref,
                wif_ref, bif_ref, whf_ref, bhf_ref,
                wir_ref, bir_ref, whr_ref, bhr_ref,
                out_ref, cp_ref):
    f32 = jnp.float32
    bb = c_ref.shape[0]
    c_sm = jnp.swapaxes(c_ref[:], 0, 1)
    cp = jnp.dot(c_sm.reshape(S * bb, D), wp_ref[:],
                 preferred_element_type=f32) + bp_ref[:]
    cp_ref[:] = cp.reshape(S, bb, HID)
    wif = wif_ref[:]
    wir = wir_ref[:]
    whf = whf_ref[:]
    whr = whr_ref[:]
    zx = jnp.zeros((HID, 64), f32)
    zh = jnp.zeros((64, 64), f32)
    wx = jnp.concatenate([
        jnp.concatenate([wif[:, 0:64], zx, wif[:, 64:128], zx,
                         wif[:, 128:192], zx], axis=1),
        jnp.concatenate([zx, wir[:, 0:64], zx, wir[:, 64:128],
                         zx, wir[:, 128:192]], axis=1)], axis=0)
    wh = jnp.concatenate([
        jnp.concatenate([whf[:, 0:64], zh, whf[:, 64:128], zh,
                         whf[:, 128:192], zh], axis=1),
        jnp.concatenate([zh, whr[:, 0:64], zh, whr[:, 64:128],
                         zh, whr[:, 128:192]], axis=1)], axis=0)
    bif = bif_ref[:]
    bir = bir_ref[:]
    bhf = bhf_ref[:]
    bhr = bhr_ref[:]
    bx = jnp.concatenate([bif[:, 0:64], bir[:, 0:64], bif[:, 64:128],
                          bir[:, 64:128], bif[:, 128:192],
                          bir[:, 128:192]], axis=1)
    bh = jnp.concatenate([bhf[:, 0:64], bhr[:, 0:64], bhf[:, 64:128],
                          bhr[:, 64:128], bhf[:, 128:192],
                          bhr[:, 128:192]], axis=1)

    def step(i, h):
        j = S - 1 - i
        xf = cp_ref[pl.ds(i, 1)].reshape(bb, HID)
        xb = cp_ref[pl.ds(j, 1)].reshape(bb, HID)
        xc = jnp.concatenate([xf, xb], axis=1)
        gx = jnp.dot(xc, wx, preferred_element_type=f32) + bx
        gh = jnp.dot(h, wh, preferred_element_type=f32) + bh
        r = jax.nn.sigmoid(gx[:, 0:128] + gh[:, 0:128])
        zg = jax.nn.sigmoid(gx[:, 128:256] + gh[:, 128:256])
        n = jnp.tanh(gx[:, 256:384] + r * gh[:, 256:384])
        h = (1.0 - zg) * n + zg * h
        out_ref[pl.ds(i, 1), :, 0:64] = h[None, :, 0:64]
        out_ref[pl.ds(j, 1), :, 64:128] = h[None, :, 64:128]
        return h

    h0 = jnp.zeros((bb, HID), f32)
    jax.lax.fori_loop(0, S, step, h0, unroll=4)


def _graph_kernel(h_ref, w1_ref, w2_ref, out_ref):
    f32 = jnp.float32
    h = jnp.swapaxes(h_ref[:], 0, 1)
    bb = h.shape[0]
    nrm = jnp.sqrt(jnp.sum(h * h, axis=-1, keepdims=True))
    hn = h / jnp.maximum(nrm, 1e-12)
    dn_sim = (((2,), (2,)), ((0,), (0,)))
    sim = jax.lax.dot_general(hn, hn, dn_sim, preferred_element_type=f32)
    simc = sim
    for _ in range(KNN):
        m = jnp.max(simc, axis=-1, keepdims=True)
        simc = jnp.where(simc == m, -3.0, simc)
    eye = (jax.lax.broadcasted_iota(jnp.int32, (bb, S, S), 2) ==
           jax.lax.broadcasted_iota(jnp.int32, (bb, S, S), 1))
    a = (simc != sim).astype(f32) + eye.astype(f32)
    deg = jnp.sum(a, axis=-1, keepdims=True) + 1e-8
    dinv = jax.lax.rsqrt(deg)
    eye = (jax.lax.broadcasted_iota(jnp.int32, (S, S), 0) ==
           jax.lax.broadcasted_iota(jnp.int32, (S, S), 1)).astype(f32)
    dinv_m = jax.lax.dot_general(
        dinv, eye, (((1,), (0,)), ((), ())),
        preferred_element_type=f32,
        precision=jax.lax.Precision.HIGHEST)
    a_norm = (dinv * a) * dinv_m
    dn_a = (((2,), (1,)), ((0,), (0,)))
    hw1 = jnp.dot(h.reshape(bb * S, HID), w1_ref[:],
                  preferred_element_type=f32).reshape(bb, S, HID)
    y1 = jax.lax.dot_general(a_norm, hw1, dn_a, preferred_element_type=f32)
    z1 = jnp.maximum(y1, 0.0)
    zw2 = jnp.dot(z1.reshape(bb * S, HID), w2_ref[:],
                  preferred_element_type=f32).reshape(bb, S, HID)
    out_ref[:] = jax.lax.dot_general(a_norm, zw2, dn_a,
                                     preferred_element_type=f32)


def kernel(C_masked, W_proj, b_proj, W_ih_f, W_hh_f, b_ih_f, b_hh_f,
           W_ih_r, W_hh_r, b_ih_r, b_hh_r, W_gcn1, W_gcn2):
    f32 = jnp.float32
    B, E = C_masked.shape
    bb1 = min(256, B)
    bb2 = min(128, B)

    C_bm = C_masked.reshape(B, S, D)
    wifT = W_ih_f.T
    wirT = W_ih_r.T
    whfT = W_hh_f.T
    whrT = W_hh_r.T

    wspec = lambda shape: pl.BlockSpec(shape, lambda i: (0, 0))
    h_sm = pl.pallas_call(
        _gru_kernel,
        grid=(B // bb1,),
        in_specs=[
            pl.BlockSpec((bb1, S, D), lambda i: (i, 0, 0)),
            wspec((D, HID)), wspec((1, HID)),
            wspec((HID, G3)), wspec((1, G3)), wspec((64, G3)), wspec((1, G3)),
            wspec((HID, G3)), wspec((1, G3)), wspec((64, G3)), wspec((1, G3)),
        ],
        out_specs=pl.BlockSpec((S, bb1, HID), lambda i: (0, i, 0)),
        out_shape=jax.ShapeDtypeStruct((S, B, HID), f32),
        scratch_shapes=[pltpu.VMEM((S, bb1, HID), f32)],
        compiler_params=pltpu.CompilerParams(
            dimension_semantics=("arbitrary",),
            vmem_limit_bytes=56 * 1024 * 1024,
        ),
    )(C_bm, W_proj, b_proj[None], wifT, b_ih_f[None], whfT, b_hh_f[None],
      wirT, b_ih_r[None], whrT, b_hh_r[None])

    z = pl.pallas_call(
        _graph_kernel,
        grid=(B // bb2,),
        in_specs=[
            pl.BlockSpec((S, bb2, HID), lambda i: (0, i, 0)),
            wspec((HID, HID)), wspec((HID, HID)),
        ],
        out_specs=pl.BlockSpec((bb2, S, HID), lambda i: (i, 0, 0)),
        out_shape=jax.ShapeDtypeStruct((B, S, HID), f32),
        compiler_params=pltpu.CompilerParams(
            dimension_semantics=("arbitrary",),
            vmem_limit_bytes=56 * 1024 * 1024,
        ),
    )(h_sm, W_gcn1, W_gcn2)
    return z


# speedup vs baseline: 5.2869x; 1.0308x over previous
import jax
import jax.numpy as jnp
import numpy as np
from jax.experimental import pallas as pl
from jax.experimental.pallas import tpu as pltpu

S = 64
D = 64
HID = 128
G3 = 192
KNN = 8


def _gru_kernel(c_ref, wp_ref, bp_ref,
                wif_ref, bif_ref, whf_ref, bhf_ref,
                wir_ref, bir_ref, whr_ref, bhr_ref,
                out_ref, cp_ref):
    f32 = jnp.float32
    bb = c_ref.shape[0]
    c_sm = jnp.swapaxes(c_ref[:], 0, 1)
    cp = jnp.dot(c_sm.reshape(S * bb, D), wp_ref[:],
                 preferred_element_type=f32) + bp_ref[:]
    cp_ref[:] = cp.reshape(S, bb, HID)
    wif = wif_ref[:]
    wir = wir_ref[:]
    whf = whf_ref[:]
    whr = whr_ref[:]
    zx = jnp.zeros((HID, 64), f32)
    zh = jnp.zeros((64, 64), f32)
    wx = jnp.concatenate([
        jnp.concatenate([wif[:, 0:64], zx, wif[:, 64:128], zx,
                         wif[:, 128:192], zx], axis=1),
        jnp.concatenate([zx, wir[:, 0:64], zx, wir[:, 64:128],
                         zx, wir[:, 128:192]], axis=1)], axis=0)
    wh = jnp.concatenate([
        jnp.concatenate([whf[:, 0:64], zh, whf[:, 64:128], zh,
                         whf[:, 128:192], zh], axis=1),
        jnp.concatenate([zh, whr[:, 0:64], zh, whr[:, 64:128],
                         zh, whr[:, 128:192]], axis=1)], axis=0)
    bif = bif_ref[:]
    bir = bir_ref[:]
    bhf = bhf_ref[:]
    bhr = bhr_ref[:]
    bx = jnp.concatenate([bif[:, 0:64], bir[:, 0:64], bif[:, 64:128],
                          bir[:, 64:128], bif[:, 128:192],
                          bir[:, 128:192]], axis=1)
    bh = jnp.concatenate([bhf[:, 0:64], bhr[:, 0:64], bhf[:, 64:128],
                          bhr[:, 64:128], bhf[:, 128:192],
                          bhr[:, 128:192]], axis=1)

    def step(i, h):
        j = S - 1 - i
        xf = cp_ref[pl.ds(i, 1)].reshape(bb, HID)
        xb = cp_ref[pl.ds(j, 1)].reshape(bb, HID)
        xc = jnp.concatenate([xf, xb], axis=1)
        gx = jnp.dot(xc, wx, preferred_element_type=f32) + bx
        gh = jnp.dot(h, wh, preferred_element_type=f32) + bh
        r = jax.nn.sigmoid(gx[:, 0:128] + gh[:, 0:128])
        zg = jax.nn.sigmoid(gx[:, 128:256] + gh[:, 128:256])
        n = jnp.tanh(gx[:, 256:384] + r * gh[:, 256:384])
        h = (1.0 - zg) * n + zg * h
        out_ref[pl.ds(i, 1), :, 0:64] = h[None, :, 0:64]
        out_ref[pl.ds(j, 1), :, 64:128] = h[None, :, 64:128]
        return h

    h0 = jnp.zeros((bb, HID), f32)
    jax.lax.fori_loop(0, S, step, h0, unroll=8)


def _graph_kernel(h_ref, w1_ref, w2_ref, out_ref):
    f32 = jnp.float32
    h = jnp.swapaxes(h_ref[:], 0, 1)
    bb = h.shape[0]
    nrm = jnp.sqrt(jnp.sum(h * h, axis=-1, keepdims=True))
    hn = h / jnp.maximum(nrm, 1e-12)
    dn_sim = (((2,), (2,)), ((0,), (0,)))
    sim = jax.lax.dot_general(hn, hn, dn_sim, preferred_element_type=f32)
    simc = sim
    for _ in range(KNN):
        m = jnp.max(simc, axis=-1, keepdims=True)
        simc = jnp.where(simc == m, -3.0, simc)
    eye = (jax.lax.broadcasted_iota(jnp.int32, (bb, S, S), 2) ==
           jax.lax.broadcasted_iota(jnp.int32, (bb, S, S), 1))
    a = (simc != sim).astype(f32) + eye.astype(f32)
    deg = jnp.sum(a, axis=-1, keepdims=True) + 1e-8
    dinv = jax.lax.rsqrt(deg)
    eye = (jax.lax.broadcasted_iota(jnp.int32, (S, S), 0) ==
           jax.lax.broadcasted_iota(jnp.int32, (S, S), 1)).astype(f32)
    dinv_m = jax.lax.dot_general(
        dinv, eye, (((1,), (0,)), ((), ())),
        preferred_element_type=f32,
        precision=jax.lax.Precision.HIGHEST)
    a_norm = (dinv * a) * dinv_m
    dn_a = (((2,), (1,)), ((0,), (0,)))
    hw1 = jnp.dot(h.reshape(bb * S, HID), w1_ref[:],
                  preferred_element_type=f32).reshape(bb, S, HID)
    y1 = jax.lax.dot_general(a_norm, hw1, dn_a, preferred_element_type=f32)
    z1 = jnp.maximum(y1, 0.0)
    zw2 = jnp.dot(z1.reshape(bb * S, HID), w2_ref[:],
                  preferred_element_type=f32).reshape(bb, S, HID)
    out_ref[:] = jax.lax.dot_general(a_norm, zw2, dn_a,
                                     preferred_element_type=f32)


def kernel(C_masked, W_proj, b_proj, W_ih_f, W_hh_f, b_ih_f, b_hh_f,
           W_ih_r, W_hh_r, b_ih_r, b_hh_r, W_gcn1, W_gcn2):
    f32 = jnp.float32
    B, E = C_masked.shape
    bb1 = min(256, B)
    bb2 = min(128, B)

    C_bm = C_masked.reshape(B, S, D)
    wifT = W_ih_f.T
    wirT = W_ih_r.T
    whfT = W_hh_f.T
    whrT = W_hh_r.T

    wspec = lambda shape: pl.BlockSpec(shape, lambda i: (0, 0))
    h_sm = pl.pallas_call(
        _gru_kernel,
        grid=(B // bb1,),
        in_specs=[
            pl.BlockSpec((bb1, S, D), lambda i: (i, 0, 0)),
            wspec((D, HID)), wspec((1, HID)),
            wspec((HID, G3)), wspec((1, G3)), wspec((64, G3)), wspec((1, G3)),
            wspec((HID, G3)), wspec((1, G3)), wspec((64, G3)), wspec((1, G3)),
        ],
        out_specs=pl.BlockSpec((S, bb1, HID), lambda i: (0, i, 0)),
        out_shape=jax.ShapeDtypeStruct((S, B, HID), f32),
        scratch_shapes=[pltpu.VMEM((S, bb1, HID), f32)],
        compiler_params=pltpu.CompilerParams(
            dimension_semantics=("arbitrary",),
            vmem_limit_bytes=56 * 1024 * 1024,
        ),
    )(C_bm, W_proj, b_proj[None], wifT, b_ih_f[None], whfT, b_hh_f[None],
      wirT, b_ih_r[None], whrT, b_hh_r[None])

    z = pl.pallas_call(
        _graph_kernel,
        grid=(B // bb2,),
        in_specs=[
            pl.BlockSpec((S, bb2, HID), lambda i: (0, i, 0)),
            wspec((HID, HID)), wspec((HID, HID)),
        ],
        out_specs=pl.BlockSpec((bb2, S, HID), lambda i: (i, 0, 0)),
        out_shape=jax.ShapeDtypeStruct((B, S, HID), f32),
        compiler_params=pltpu.CompilerParams(
            dimension_semantics=("arbitrary",),
            vmem_limit_bytes=56 * 1024 * 1024,
        ),
    )(h_sm, W_gcn1, W_gcn2)
    return z
